# baseline (speedup 1.0000x reference)
import functools

import jax
import jax.numpy as jnp
from jax import lax
from jax.experimental import pallas as pl
from jax.experimental.pallas import tpu as pltpu
from jax.experimental.pallas import tpu_sc as plsc

EMBED_DIM = 32
NUM_GENRES = 18
WPAD = 96


def _make_sc_kernel(B, n_workers):
    rows_per_w = B // n_workers
    n_chunks = rows_per_w // 128
    n_groups = rows_per_w // 16
    groups_per_blk = 8
    n_blocks = n_groups // groups_per_blk

    mesh = plsc.VectorSubcoreMesh(core_axis_name="c", subcore_axis_name="s")

    @functools.partial(
        pl.kernel,
        mesh=mesh,
        out_type=jax.ShapeDtypeStruct((B,), jnp.float32),
        compiler_params=pltpu.CompilerParams(
            use_tc_tiling_on_sc=False, needs_layout_passes=False),
        scratch_types=[
            pltpu.VMEM((rows_per_w,), jnp.int32),
            pltpu.VMEM((rows_per_w,), jnp.int32),
            pltpu.VMEM((rows_per_w, EMBED_DIM), jnp.float32),
            pltpu.VMEM((rows_per_w, EMBED_DIM), jnp.float32),
            pltpu.VMEM((rows_per_w,), jnp.float32),
            pltpu.VMEM((rows_per_w,), jnp.float32),
            pltpu.VMEM((rows_per_w,), jnp.float32),
            pltpu.VMEM((rows_per_w, NUM_GENRES), jnp.float32),
            pltpu.VMEM((WPAD, 16), jnp.float32),
            pltpu.VMEM((rows_per_w,), jnp.float32),
            pltpu.SemaphoreType.DMA,
        ],
    )
    def body(ut_hbm, mt_hbm, uids_hbm, mids_hbm, gen_hbm, age_hbm, occ_hbm,
             genres_hbm, wb_hbm, out_hbm,
             uidx_v, midx_v, urows_v, mrows_v, gen_v, age_v, occ_v,
             genres_v, wb_v, out_v, sem):
        wid = lax.axis_index("s") * 2 + lax.axis_index("c")
        base = wid * rows_per_w

        pltpu.sync_copy(uids_hbm.at[pl.ds(base, rows_per_w)], uidx_v)
        pltpu.sync_copy(mids_hbm.at[pl.ds(base, rows_per_w)], midx_v)

        copies = []
        for k in range(n_chunks):
            sl = pl.ds(k * 128, 128)
            copies.append(pltpu.async_copy(
                ut_hbm.at[uidx_v.at[sl]], urows_v.at[sl], sem))
            copies.append(pltpu.async_copy(
                mt_hbm.at[midx_v.at[sl]], mrows_v.at[sl], sem))
        copies.append(pltpu.async_copy(
            gen_hbm.at[pl.ds(base, rows_per_w)], gen_v, sem))
        copies.append(pltpu.async_copy(
            age_hbm.at[pl.ds(base, rows_per_w)], age_v, sem))
        copies.append(pltpu.async_copy(
            occ_hbm.at[pl.ds(base, rows_per_w)], occ_v, sem))
        copies.append(pltpu.async_copy(
            genres_hbm.at[pl.ds(base, rows_per_w)], genres_v, sem))
        copies.append(pltpu.async_copy(wb_hbm, wb_v, sem))
        for c in copies:
            c.wait()

        iota = lax.iota(jnp.int32, 16)
        wg_splat = wb_v[82]
        wa_splat = wb_v[83]
        wo_splat = wb_v[84]

        def block_body(blk, carry):
            rows0 = blk * (groups_per_blk * 16)
            row_vecs = [rows0 + g * 16 + iota for g in range(groups_per_blk)]
            accs = []
            for g in range(groups_per_blk):
                off = rows0 + g * 16
                acc = (gen_v[pl.ds(off, 16)] * wg_splat
                       + age_v[pl.ds(off, 16)] * wa_splat
                       + occ_v[pl.ds(off, 16)] * wo_splat)
                accs.append(acc)
            for j in range(EMBED_DIM):
                wj = wb_v[j]
                col = jnp.full((16,), j, jnp.int32)
                for g in range(groups_per_blk):
                    accs[g] = accs[g] + plsc.load_gather(
                        urows_v, [row_vecs[g], col]) * wj
            for j in range(EMBED_DIM):
                wj = wb_v[EMBED_DIM + j]
                col = jnp.full((16,), j, jnp.int32)
                for g in range(groups_per_blk):
                    accs[g] = accs[g] + plsc.load_gather(
                        mrows_v, [row_vecs[g], col]) * wj
            for j in range(NUM_GENRES):
                wj = wb_v[64 + j]
                col = jnp.full((16,), j, jnp.int32)
                for g in range(groups_per_blk):
                    accs[g] = accs[g] + plsc.load_gather(
                        genres_v, [row_vecs[g], col]) * wj
            for g in range(groups_per_blk):
                p = 1.0 / (1.0 + jnp.exp(-accs[g]))
                out_v[pl.ds(rows0 + g * 16, 16)] = p
            return carry

        lax.fori_loop(0, n_blocks, block_body, 0)

        pltpu.sync_copy(out_v, out_hbm.at[pl.ds(base, rows_per_w)])

    return body


def kernel(user_ids, movie_ids, gender, age, occupation, genres,
           user_table, movie_table, W):
    B = user_ids.shape[0]
    w_row = W[0].astype(jnp.float32)
    w_pad = jnp.concatenate([
        w_row[:64],
        w_row[67:85],
        w_row[64:67],
        jnp.zeros((WPAD - 85,), jnp.float32),
    ])
    w_bcast = jnp.broadcast_to(w_pad[:, None], (WPAD, 16))

    sc_kernel = _make_sc_kernel(B, 32)
    return sc_kernel(user_table, movie_table,
                     user_ids.astype(jnp.int32), movie_ids.astype(jnp.int32),
                     gender, age, occupation, genres, w_bcast)


# speedup vs baseline: 1.0093x; 1.0093x over previous
import functools

import jax
import jax.numpy as jnp
from jax import lax
from jax.experimental import pallas as pl
from jax.experimental.pallas import tpu as pltpu
from jax.experimental.pallas import tpu_sc as plsc

EMBED_DIM = 32
NUM_GENRES = 18
WPAD = 96
ROWS_PER_PAD = 128 // EMBED_DIM
CHUNK = 128


def _make_sc_kernel(B, n_users, n_movies, n_workers):
    rows_per_w = B // n_workers
    n_chunks = rows_per_w // CHUNK
    groups_per_chunk = CHUNK // 16

    mesh = plsc.VectorSubcoreMesh(core_axis_name="c", subcore_axis_name="s")

    @functools.partial(
        pl.kernel,
        mesh=mesh,
        out_type=jax.ShapeDtypeStruct((B,), jnp.float32),
        compiler_params=pltpu.CompilerParams(
            use_tc_tiling_on_sc=False, needs_layout_passes=False),
        scratch_types=[
            pltpu.VMEM((rows_per_w,), jnp.int32),
            pltpu.VMEM((rows_per_w,), jnp.int32),
            pltpu.VMEM((CHUNK, 128), jnp.float32),
            pltpu.VMEM((CHUNK, 128), jnp.float32),
            pltpu.VMEM((rows_per_w,), jnp.float32),
            pltpu.VMEM((rows_per_w,), jnp.float32),
            pltpu.VMEM((rows_per_w,), jnp.float32),
            pltpu.VMEM((rows_per_w * NUM_GENRES,), jnp.float32),
            pltpu.VMEM((WPAD, 16), jnp.float32),
            pltpu.VMEM((rows_per_w,), jnp.float32),
            pltpu.SemaphoreType.DMA,
        ],
    )
    def body(ut_hbm, mt_hbm, uids_hbm, mids_hbm, gen_hbm, age_hbm, occ_hbm,
             genres_hbm, wb_hbm, out_hbm,
             uidx_v, midx_v, urows_v, mrows_v,
             gen_v, age_v, occ_v, genres_v, wb_v, out_v, sem):
        wid = lax.axis_index("s") * 2 + lax.axis_index("c")
        base = wid * rows_per_w

        pltpu.sync_copy(uids_hbm.at[pl.ds(base, rows_per_w)], uidx_v)
        pltpu.sync_copy(mids_hbm.at[pl.ds(base, rows_per_w)], midx_v)

        copies = [
            pltpu.async_copy(gen_hbm.at[pl.ds(base, rows_per_w)], gen_v, sem),
            pltpu.async_copy(age_hbm.at[pl.ds(base, rows_per_w)], age_v, sem),
            pltpu.async_copy(occ_hbm.at[pl.ds(base, rows_per_w)], occ_v, sem),
            pltpu.async_copy(
                genres_hbm.at[pl.ds(base * NUM_GENRES,
                                    rows_per_w * NUM_GENRES)], genres_v, sem),
            pltpu.async_copy(wb_hbm, wb_v, sem),
        ]
        for c in copies:
            c.wait()

        iota = lax.iota(jnp.int32, 16)
        wg_splat = wb_v[82]
        wa_splat = wb_v[83]
        wo_splat = wb_v[84]

        def chunk_body(k, _):
            ch = pl.ds(k * CHUNK, CHUNK)
            cu = pltpu.async_copy(ut_hbm.at[uidx_v.at[ch]], urows_v, sem)
            cm = pltpu.async_copy(mt_hbm.at[midx_v.at[ch]], mrows_v, sem)
            cu.wait()
            cm.wait()

            def group_body(g, _):
                off = k * CHUNK + g * 16
                p16 = g * 16 + iota
                acc = (gen_v[pl.ds(off, 16)] * wg_splat
                       + age_v[pl.ds(off, 16)] * wa_splat
                       + occ_v[pl.ds(off, 16)] * wo_splat)
                for j in range(EMBED_DIM):
                    col = jnp.full((16,), j, jnp.int32)
                    acc = acc + plsc.load_gather(
                        urows_v, [p16, col]) * wb_v[j]
                for j in range(EMBED_DIM):
                    col = jnp.full((16,), j, jnp.int32)
                    acc = acc + plsc.load_gather(
                        mrows_v, [p16, col]) * wb_v[EMBED_DIM + j]
                gflat = (off + iota) * NUM_GENRES
                for j in range(NUM_GENRES):
                    acc = acc + plsc.load_gather(
                        genres_v, [gflat + j]) * wb_v[64 + j]
                out_v[pl.ds(off, 16)] = 1.0 / (1.0 + jnp.exp(-acc))
                return 0

            lax.fori_loop(0, groups_per_chunk, group_body, 0)
            return 0

        lax.fori_loop(0, n_chunks, chunk_body, 0)

        pltpu.sync_copy(out_v, out_hbm.at[pl.ds(base, rows_per_w)])

    return body


def kernel(user_ids, movie_ids, gender, age, occupation, genres,
           user_table, movie_table, W):
    B = user_ids.shape[0]
    n_users, d = user_table.shape
    n_movies, _ = movie_table.shape
    ut128 = jnp.pad(user_table, ((0, 0), (0, 128 - d)))
    mt128 = jnp.pad(movie_table, ((0, 0), (0, 128 - d)))
    genres1d = genres.reshape(-1)

    w_row = W[0].astype(jnp.float32)
    w_pad = jnp.concatenate([
        w_row[:64],
        w_row[67:85],
        w_row[64:67],
        jnp.zeros((WPAD - 85,), jnp.float32),
    ])
    w_bcast = jnp.broadcast_to(w_pad[:, None], (WPAD, 16))

    sc_kernel = _make_sc_kernel(B, n_users, n_movies, 32)
    return sc_kernel(ut128, mt128,
                     user_ids.astype(jnp.int32), movie_ids.astype(jnp.int32),
                     gender, age, occupation, genres1d, w_bcast)


# speedup vs baseline: 3.5774x; 3.5445x over previous
import functools

import jax
import jax.numpy as jnp
from jax import lax
from jax.experimental import pallas as pl
from jax.experimental.pallas import tpu as pltpu
from jax.experimental.pallas import tpu_sc as plsc

EMBED_DIM = 32
NUM_GENRES = 18
GPAD = 24
WPAD = 96
CHUNK = 128


def _tiles(n):
    return (n + 127) // 128


def _make_sc_kernel(B, n_users, n_movies, n_workers):
    rows_per_w = B // n_workers
    n_elems = rows_per_w * EMBED_DIM
    n_streams = n_elems // CHUNK
    ut_tiles = _tiles(n_users)
    mt_tiles = _tiles(n_movies)
    gb_per_w = rows_per_w // 128

    uoff = [((c // 8) * ut_tiles) * 1024 + (c % 8) * 128
            for c in range(EMBED_DIM)]
    moff = [((c // 8) * mt_tiles) * 1024 + (c % 8) * 128
            for c in range(EMBED_DIM)]

    mesh = plsc.VectorSubcoreMesh(core_axis_name="c", subcore_axis_name="s")

    @functools.partial(
        pl.kernel,
        mesh=mesh,
        out_type=jax.ShapeDtypeStruct((B,), jnp.float32),
        compiler_params=pltpu.CompilerParams(
            use_tc_tiling_on_sc=False, needs_layout_passes=False),
        scratch_types=[
            pltpu.VMEM((rows_per_w,), jnp.int32),
            pltpu.VMEM((rows_per_w,), jnp.int32),
            pltpu.VMEM((n_elems,), jnp.int32),
            pltpu.VMEM((n_elems,), jnp.int32),
            pltpu.VMEM((n_elems,), jnp.float32),
            pltpu.VMEM((n_elems,), jnp.float32),
            pltpu.VMEM((rows_per_w,), jnp.float32),
            pltpu.VMEM((rows_per_w,), jnp.float32),
            pltpu.VMEM((rows_per_w,), jnp.float32),
            pltpu.VMEM((GPAD // 8, 4, 8, 128), jnp.float32),
            pltpu.VMEM((WPAD, 16), jnp.float32),
            pltpu.VMEM((rows_per_w,), jnp.float32),
            pltpu.SemaphoreType.DMA,
        ],
    )
    def body(ut_hbm, mt_hbm, uids_hbm, mids_hbm, gen_hbm, age_hbm, occ_hbm,
             genres4_hbm, wb_hbm, out_hbm,
             uidx_v, midx_v, uflat_v, mflat_v, ucols_v, mcols_v,
             gen_v, age_v, occ_v, genres4_v, wb_v, out_v, sem):
        wid = lax.axis_index("s") * 2 + lax.axis_index("c")
        base = wid * rows_per_w

        pltpu.sync_copy(uids_hbm.at[pl.ds(base, rows_per_w)], uidx_v)
        pltpu.sync_copy(mids_hbm.at[pl.ds(base, rows_per_w)], midx_v)

        dense_copies = [
            pltpu.async_copy(gen_hbm.at[pl.ds(base, rows_per_w)], gen_v, sem),
            pltpu.async_copy(age_hbm.at[pl.ds(base, rows_per_w)], age_v, sem),
            pltpu.async_copy(occ_hbm.at[pl.ds(base, rows_per_w)], occ_v, sem),
            pltpu.async_copy(
                genres4_hbm.at[:, pl.ds(wid * gb_per_w, gb_per_w)],
                genres4_v, sem),
            pltpu.async_copy(wb_hbm, wb_v, sem),
        ]

        def idx_body(g, _):
            sl = pl.ds(g * 16, 16)
            u = uidx_v[sl]
            m = midx_v[sl]
            ub = (lax.shift_right_logical(u, 7) * 1024
                  + lax.bitwise_and(u, 127))
            mb = (lax.shift_right_logical(m, 7) * 1024
                  + lax.bitwise_and(m, 127))
            for c in range(EMBED_DIM):
                uflat_v[pl.ds(c * rows_per_w + g * 16, 16)] = ub + uoff[c]
                mflat_v[pl.ds(c * rows_per_w + g * 16, 16)] = mb + moff[c]
            return 0

        lax.fori_loop(0, rows_per_w // 16, idx_body, 0)

        gathers = []
        for t in range(n_streams):
            sl = pl.ds(t * CHUNK, CHUNK)
            gathers.append(pltpu.async_copy(
                ut_hbm.at[uflat_v.at[sl]], ucols_v.at[sl], sem))
            gathers.append(pltpu.async_copy(
                mt_hbm.at[mflat_v.at[sl]], mcols_v.at[sl], sem))
        for c in dense_copies:
            c.wait()
        for c in gathers:
            c.wait()

        wg_splat = wb_v[82]
        wa_splat = wb_v[83]
        wo_splat = wb_v[84]

        def group_body(g, _):
            off = g * 16
            acc = (gen_v[pl.ds(off, 16)] * wg_splat
                   + age_v[pl.ds(off, 16)] * wa_splat
                   + occ_v[pl.ds(off, 16)] * wo_splat)
            for c in range(EMBED_DIM):
                acc = acc + ucols_v[pl.ds(c * rows_per_w + off, 16)] * wb_v[c]
            for c in range(EMBED_DIM):
                acc = (acc + mcols_v[pl.ds(c * rows_per_w + off, 16)]
                       * wb_v[EMBED_DIM + c])
            b = g // 8
            l0 = (g % 8) * 16
            for c in range(NUM_GENRES):
                acc = (acc + genres4_v[c // 8, b, c % 8, pl.ds(l0, 16)]
                       * wb_v[64 + c])
            out_v[pl.ds(off, 16)] = 1.0 / (1.0 + jnp.exp(-acc))
            return 0

        lax.fori_loop(0, rows_per_w // 16, group_body, 0)

        pltpu.sync_copy(out_v, out_hbm.at[pl.ds(base, rows_per_w)])

    return body


def _native_flat_alias(table):
    n, d = table.shape
    nt = _tiles(n)
    p = jnp.pad(table, ((0, nt * 128 - n), (0, 0)))
    x = p.T.reshape(d // 8, 8, nt, 128).transpose(0, 2, 1, 3)
    return x.reshape(-1)


def kernel(user_ids, movie_ids, gender, age, occupation, genres,
           user_table, movie_table, W):
    B = user_ids.shape[0]
    n_users, d = user_table.shape
    n_movies, _ = movie_table.shape

    ut1d = _native_flat_alias(user_table)
    mt1d = _native_flat_alias(movie_table)

    gp = jnp.pad(genres, ((0, 0), (0, GPAD - NUM_GENRES)))
    genres4 = gp.T.reshape(GPAD // 8, 8, B // 128, 128).transpose(0, 2, 1, 3)

    w_row = W[0].astype(jnp.float32)
    w_pad = jnp.concatenate([
        w_row[:64],
        w_row[67:85],
        w_row[64:67],
        jnp.zeros((WPAD - 85,), jnp.float32),
    ])
    w_bcast = jnp.broadcast_to(w_pad[:, None], (WPAD, 16))

    sc_kernel = _make_sc_kernel(B, n_users, n_movies, 32)
    return sc_kernel(ut1d, mt1d,
                     user_ids.astype(jnp.int32), movie_ids.astype(jnp.int32),
                     gender, age, occupation, genres4, w_bcast)


# speedup vs baseline: 3.8680x; 1.0812x over previous
import functools

import jax
import jax.numpy as jnp
from jax import lax
from jax.experimental import pallas as pl
from jax.experimental.pallas import tpu as pltpu
from jax.experimental.pallas import tpu_sc as plsc

EMBED_DIM = 32
NUM_GENRES = 18
GPAD = 24
WPAD = 96
CHUNK = 128
N_WORKERS = 32


def _tiles(n):
    return (n + 127) // 128


def _feature_offsets(n_rows):
    nt = _tiles(n_rows)
    return [((c // 8) * nt) * 1024 + (c % 8) * 128 for c in range(EMBED_DIM)]


def _mesh():
    return plsc.VectorSubcoreMesh(core_axis_name="c", subcore_axis_name="s")


def _make_kernel_a(B, n_movies):
    rows_per_w = B // N_WORKERS
    n_elems = rows_per_w * EMBED_DIM
    n_streams = n_elems // CHUNK
    gb_per_w = rows_per_w // 128
    moff = _feature_offsets(n_movies)

    @functools.partial(
        pl.kernel,
        mesh=_mesh(),
        out_type=jax.ShapeDtypeStruct((B,), jnp.float32),
        compiler_params=pltpu.CompilerParams(
            use_tc_tiling_on_sc=False, needs_layout_passes=False),
        scratch_types=[
            pltpu.VMEM((rows_per_w,), jnp.int32),
            pltpu.VMEM((n_elems,), jnp.int32),
            pltpu.VMEM((n_elems,), jnp.float32),
            pltpu.VMEM((rows_per_w,), jnp.float32),
            pltpu.VMEM((rows_per_w,), jnp.float32),
            pltpu.VMEM((rows_per_w,), jnp.float32),
            pltpu.VMEM((GPAD // 8, 4, 8, 128), jnp.float32),
            pltpu.VMEM((WPAD, 16), jnp.float32),
            pltpu.VMEM((rows_per_w,), jnp.float32),
            pltpu.SemaphoreType.DMA,
        ],
    )
    def body_a(mt_hbm, mids_hbm, gen_hbm, age_hbm, occ_hbm, genres4_hbm,
               wb_hbm, out_hbm,
               midx_v, mflat_v, mcols_v, gen_v, age_v, occ_v, genres4_v,
               wb_v, out_v, sem):
        wid = lax.axis_index("s") * 2 + lax.axis_index("c")
        base = wid * rows_per_w

        pltpu.sync_copy(mids_hbm.at[pl.ds(base, rows_per_w)], midx_v)

        dense_copies = [
            pltpu.async_copy(gen_hbm.at[pl.ds(base, rows_per_w)], gen_v, sem),
            pltpu.async_copy(age_hbm.at[pl.ds(base, rows_per_w)], age_v, sem),
            pltpu.async_copy(occ_hbm.at[pl.ds(base, rows_per_w)], occ_v, sem),
            pltpu.async_copy(
                genres4_hbm.at[:, pl.ds(wid * gb_per_w, gb_per_w)],
                genres4_v, sem),
            pltpu.async_copy(wb_hbm, wb_v, sem),
        ]

        def idx_body(g, _):
            m = midx_v[pl.ds(g * 16, 16)]
            mb = (lax.shift_right_logical(m, 7) * 1024
                  + lax.bitwise_and(m, 127))
            for c in range(EMBED_DIM):
                mflat_v[pl.ds(c * rows_per_w + g * 16, 16)] = mb + moff[c]
            return 0

        lax.fori_loop(0, rows_per_w // 16, idx_body, 0)

        gathers = []
        for t in range(n_streams):
            sl = pl.ds(t * CHUNK, CHUNK)
            gathers.append(pltpu.async_copy(
                mt_hbm.at[mflat_v.at[sl]], mcols_v.at[sl], sem))
        for c in dense_copies:
            c.wait()
        for c in gathers:
            c.wait()

        wg_splat = wb_v[82]
        wa_splat = wb_v[83]
        wo_splat = wb_v[84]

        def group_body(g, _):
            off = g * 16
            acc = (gen_v[pl.ds(off, 16)] * wg_splat
                   + age_v[pl.ds(off, 16)] * wa_splat
                   + occ_v[pl.ds(off, 16)] * wo_splat)
            for c in range(EMBED_DIM):
                acc = (acc + mcols_v[pl.ds(c * rows_per_w + off, 16)]
                       * wb_v[EMBED_DIM + c])
            b = g // 8
            l0 = (g % 8) * 16
            for c in range(NUM_GENRES):
                acc = (acc + genres4_v[c // 8, b, c % 8, pl.ds(l0, 16)]
                       * wb_v[64 + c])
            out_v[pl.ds(off, 16)] = acc
            return 0

        lax.fori_loop(0, rows_per_w // 16, group_body, 0)

        pltpu.sync_copy(out_v, out_hbm.at[pl.ds(base, rows_per_w)])

    return body_a


def _make_kernel_b(B, n_users):
    rows_per_w = B // N_WORKERS
    n_elems = rows_per_w * EMBED_DIM
    n_streams = n_elems // CHUNK
    uoff = _feature_offsets(n_users)

    @functools.partial(
        pl.kernel,
        mesh=_mesh(),
        out_type=jax.ShapeDtypeStruct((B,), jnp.float32),
        compiler_params=pltpu.CompilerParams(
            use_tc_tiling_on_sc=False, needs_layout_passes=False),
        scratch_types=[
            pltpu.VMEM((rows_per_w,), jnp.int32),
            pltpu.VMEM((n_elems,), jnp.int32),
            pltpu.VMEM((n_elems,), jnp.float32),
            pltpu.VMEM((rows_per_w,), jnp.float32),
            pltpu.VMEM((WPAD, 16), jnp.float32),
            pltpu.VMEM((rows_per_w,), jnp.float32),
            pltpu.SemaphoreType.DMA,
        ],
    )
    def body_b(ut_hbm, uids_hbm, part_hbm, wb_hbm, out_hbm,
               uidx_v, uflat_v, ucols_v, part_v, wb_v, out_v, sem):
        wid = lax.axis_index("s") * 2 + lax.axis_index("c")
        base = wid * rows_per_w

        pltpu.sync_copy(uids_hbm.at[pl.ds(base, rows_per_w)], uidx_v)

        dense_copies = [
            pltpu.async_copy(part_hbm.at[pl.ds(base, rows_per_w)], part_v,
                             sem),
            pltpu.async_copy(wb_hbm, wb_v, sem),
        ]

        def idx_body(g, _):
            u = uidx_v[pl.ds(g * 16, 16)]
            ub = (lax.shift_right_logical(u, 7) * 1024
                  + lax.bitwise_and(u, 127))
            for c in range(EMBED_DIM):
                uflat_v[pl.ds(c * rows_per_w + g * 16, 16)] = ub + uoff[c]
            return 0

        lax.fori_loop(0, rows_per_w // 16, idx_body, 0)

        gathers = []
        for t in range(n_streams):
            sl = pl.ds(t * CHUNK, CHUNK)
            gathers.append(pltpu.async_copy(
                ut_hbm.at[uflat_v.at[sl]], ucols_v.at[sl], sem))
        for c in dense_copies:
            c.wait()
        for c in gathers:
            c.wait()

        def group_body(g, _):
            off = g * 16
            acc = part_v[pl.ds(off, 16)]
            for c in range(EMBED_DIM):
                acc = acc + ucols_v[pl.ds(c * rows_per_w + off, 16)] * wb_v[c]
            out_v[pl.ds(off, 16)] = 1.0 / (1.0 + jnp.exp(-acc))
            return 0

        lax.fori_loop(0, rows_per_w // 16, group_body, 0)

        pltpu.sync_copy(out_v, out_hbm.at[pl.ds(base, rows_per_w)])

    return body_b


def _native_flat_alias(table):
    n, d = table.shape
    nt = _tiles(n)
    p = jnp.pad(table, ((0, nt * 128 - n), (0, 0)))
    x = p.T.reshape(d // 8, 8, nt, 128).transpose(0, 2, 1, 3)
    return x.reshape(-1)


def kernel(user_ids, movie_ids, gender, age, occupation, genres,
           user_table, movie_table, W):
    B = user_ids.shape[0]
    n_users, d = user_table.shape
    n_movies, _ = movie_table.shape

    ut1d = _native_flat_alias(user_table)
    mt1d = _native_flat_alias(movie_table)

    gp = jnp.pad(genres, ((0, 0), (0, GPAD - NUM_GENRES)))
    genres4 = gp.T.reshape(GPAD // 8, 8, B // 128, 128).transpose(0, 2, 1, 3)

    w_row = W[0].astype(jnp.float32)
    w_pad = jnp.concatenate([
        w_row[:64],
        w_row[67:85],
        w_row[64:67],
        jnp.zeros((WPAD - 85,), jnp.float32),
    ])
    w_bcast = jnp.broadcast_to(w_pad[:, None], (WPAD, 16))

    kernel_a = _make_kernel_a(B, n_movies)
    kernel_b = _make_kernel_b(B, n_users)
    partial = kernel_a(mt1d, movie_ids.astype(jnp.int32), gender, age,
                       occupation, genres4, w_bcast)
    return kernel_b(ut1d, user_ids.astype(jnp.int32), partial, w_bcast)
